# Initial kernel scaffold; baseline (speedup 1.0000x reference)
import jax
import jax.numpy as jnp
from jax.experimental import pallas as pl


def kernel(image_grid, query_points, query_colors, intrinsics):
    raise NotImplementedError("write your pallas kernel here")


# speedup vs baseline: 1.2224x; 1.2224x over previous
import functools

import jax
import jax.numpy as jnp
from jax import lax
from jax.experimental import pallas as pl
from jax.experimental.pallas import tpu as pltpu

_K = 5
_EPS = 1e-7
_LANE = 128
_PAD_SENTINEL = 1e18


def _round_up(x, m):
    return (x + m - 1) // m * m


def _knn_splat_kernel(coords_ref, colors_ref, out_ref, *, tp, width, n_pad):
    lin = pl.program_id(1) * tp + lax.broadcasted_iota(jnp.int32, (1, tp), 1)
    py_i = lax.div(lin, width)
    px_i = lin - py_i * width
    px = px_i.astype(jnp.float32)
    py = py_i.astype(jnp.float32)

    coords = coords_ref[0]
    qx = coords[:, 0:1]
    qy = coords[:, 1:2]

    dx = px - qx
    dy = py - qy
    d = dx * dx + dy * dy

    row = lax.broadcasted_iota(jnp.int32, (n_pad, tp), 0)
    big = jnp.int32(n_pad + 1)
    inf = jnp.float32(jnp.inf)

    for _ in range(_K):
        m = jnp.min(d, axis=0, keepdims=True)
        idx = jnp.min(jnp.where(d == m, row, big),
                      axis=0, keepdims=True)
        d = jnp.where(row == idx, inf, d)

    w = (d == inf).astype(jnp.float32)
    cols = colors_ref[0]
    acc = lax.dot_general(cols, w, (((1,), (0,)), ((), ())),
                          preferred_element_type=jnp.float32)
    out_ref[0] = (acc * (1.0 / _K)).astype(out_ref.dtype)


def kernel(image_grid, query_points, query_colors, intrinsics):
    B, C, H, W = image_grid.shape
    P = H * W
    N = query_points.shape[1]

    intr = jnp.asarray(intrinsics, jnp.float32)
    pc = query_points.astype(jnp.float32) @ intr.T
    qg = pc[..., :2] / (pc[..., 2:3] + _EPS)

    n_pad = _round_up(N, _LANE)
    if n_pad != N:
        coords = jnp.full((B, n_pad, 2), _PAD_SENTINEL, jnp.float32)
        coords = coords.at[:, :N, :].set(qg)
        colors_t = jnp.zeros((B, 3, n_pad), jnp.float32)
        colors_t = colors_t.at[:, :, :N].set(
            jnp.transpose(query_colors.astype(jnp.float32), (0, 2, 1)))
    else:
        coords = qg.astype(jnp.float32)
        colors_t = jnp.transpose(query_colors.astype(jnp.float32), (0, 2, 1))

    tp = 512
    p_pad = _round_up(P, tp)
    p_tiles = p_pad // tp

    out = pl.pallas_call(
        functools.partial(_knn_splat_kernel, tp=tp, width=W, n_pad=n_pad),
        out_shape=jax.ShapeDtypeStruct((B, 3, p_pad), image_grid.dtype),
        grid=(B, p_tiles),
        in_specs=[
            pl.BlockSpec((1, n_pad, 2), lambda b, p: (b, 0, 0)),
            pl.BlockSpec((1, 3, n_pad), lambda b, p: (b, 0, 0)),
        ],
        out_specs=pl.BlockSpec((1, 3, tp), lambda b, p: (b, 0, p)),
        compiler_params=pltpu.CompilerParams(
            dimension_semantics=("parallel", "parallel"),
            vmem_limit_bytes=64 * 1024 * 1024),
    )(coords, colors_t)

    if p_pad != P:
        out = out[:, :, :P]
    return out.reshape(B, 3, H, W)
